# baseline (speedup 1.0000x reference)
import jax
import jax.numpy as jnp
from jax import lax
from jax.experimental import pallas as pl
from jax.experimental.pallas import tpu as pltpu
from jax.experimental.pallas import tpu_sc as plsc

B = 16384
D = 128
L = 16

_info = plsc.get_sparse_core_info()
_NC = _info.num_cores
_NW = _NC * _info.num_subcores
ROWS_PER_W = B // _NW
CHUNK = 64
NCH = ROWS_PER_W // CHUNK
RPI = 4


def _sc_body(x_hbm, a_hbm, b_hbm, out_hbm, idx_a, idx_b, buf, sem_i, sem_g, sem_w):
    wid = lax.axis_index("s") * _NC + lax.axis_index("c")
    base = wid * ROWS_PER_W

    ci_a = pltpu.async_copy(a_hbm.at[pl.ds(base, ROWS_PER_W)], idx_a, sem_i)
    ci_b = pltpu.async_copy(b_hbm.at[pl.ds(base, ROWS_PER_W)], idx_b, sem_i)
    ci_a.wait()
    ga = [pltpu.async_copy(x_hbm.at[idx_a.at[pl.ds(c * CHUNK, CHUNK)]],
                           buf.at[c], sem_g.at[c])
          for c in range(NCH)]
    ci_b.wait()
    gb = [None] * NCH
    wb = [None] * NCH

    for c in range(NCH):
        ga[c].wait()

        def rows(t, carry):
            for r in range(RPI):
                for j in range(D // L):
                    i = t * RPI + r
                    sl = pl.ds(j * L, L)
                    buf[c, i, sl] = -buf[c, i, sl]
            return carry

        lax.fori_loop(0, CHUNK // RPI, rows, 0)

        gb[c] = pltpu.async_copy(x_hbm.at[idx_b.at[pl.ds(c * CHUNK, CHUNK)]],
                                 buf.at[c], sem_g.at[c], add=True)
        if c > 0:
            gb[c - 1].wait()
            wb[c - 1] = pltpu.async_copy(
                buf.at[c - 1], out_hbm.at[pl.ds(base + (c - 1) * CHUNK, CHUNK)],
                sem_w.at[c - 1])

    gb[NCH - 1].wait()
    wb[NCH - 1] = pltpu.async_copy(
        buf.at[NCH - 1], out_hbm.at[pl.ds(base + (NCH - 1) * CHUNK, CHUNK)],
        sem_w.at[NCH - 1])
    for c in range(NCH):
        wb[c].wait()


def kernel(X, pref_a, pref_b):
    mesh = plsc.VectorSubcoreMesh(core_axis_name="c", subcore_axis_name="s")
    k = pl.kernel(
        _sc_body,
        out_type=jax.ShapeDtypeStruct((B, D), jnp.float32),
        mesh=mesh,
        scratch_types=[
            pltpu.VMEM((ROWS_PER_W,), jnp.int32),
            pltpu.VMEM((ROWS_PER_W,), jnp.int32),
            pltpu.VMEM((NCH, CHUNK, D), jnp.float32),
            pltpu.SemaphoreType.DMA,
            pltpu.SemaphoreType.DMA((NCH,)),
            pltpu.SemaphoreType.DMA((NCH,)),
        ],
    )
    return k(X, pref_a.astype(jnp.int32), pref_b.astype(jnp.int32))


# speedup vs baseline: 1.0190x; 1.0190x over previous
import jax
import jax.numpy as jnp
from jax import lax
from jax.experimental import pallas as pl
from jax.experimental.pallas import tpu as pltpu
from jax.experimental.pallas import tpu_sc as plsc

B = 16384
D = 128
L = 16

_info = plsc.get_sparse_core_info()
_NC = _info.num_cores
_NW = _NC * _info.num_subcores
ROWS_PER_W = B // _NW
CHUNKS = (320, 192)
NCH = len(CHUNKS)
OFFS = (0, 320)
RPI = 2


def _sc_body(x_hbm, a_hbm, b_hbm, out_hbm, idx_a, idx_b, buf0, buf1,
             sem_i, sem_g, sem_w):
    bufs = (buf0, buf1)
    wid = lax.axis_index("s") * _NC + lax.axis_index("c")
    base = wid * ROWS_PER_W

    ci_a = pltpu.async_copy(a_hbm.at[pl.ds(base, ROWS_PER_W)], idx_a, sem_i)
    ci_b = pltpu.async_copy(b_hbm.at[pl.ds(base, ROWS_PER_W)], idx_b, sem_i)
    ci_a.wait()
    ga = [pltpu.async_copy(x_hbm.at[idx_a.at[pl.ds(OFFS[c], CHUNKS[c])]],
                           bufs[c], sem_g.at[c])
          for c in range(NCH)]
    ci_b.wait()

    gb = [None] * NCH
    wb = [None] * NCH
    for c in range(NCH):
        ga[c].wait()

        def rows(t, carry, c=c):
            for r in range(RPI):
                for j in range(D // L):
                    i = t * RPI + r
                    sl = pl.ds(j * L, L)
                    bufs[c][i, sl] = -bufs[c][i, sl]
            return carry

        lax.fori_loop(0, CHUNKS[c] // RPI, rows, 0)

        gb[c] = pltpu.async_copy(x_hbm.at[idx_b.at[pl.ds(OFFS[c], CHUNKS[c])]],
                                 bufs[c], sem_g.at[c], add=True)
        if c > 0:
            gb[c - 1].wait()
            wb[c - 1] = pltpu.async_copy(
                bufs[c - 1], out_hbm.at[pl.ds(base + OFFS[c - 1], CHUNKS[c - 1])],
                sem_w.at[c - 1])

    gb[NCH - 1].wait()
    wb[NCH - 1] = pltpu.async_copy(
        bufs[NCH - 1], out_hbm.at[pl.ds(base + OFFS[NCH - 1], CHUNKS[NCH - 1])],
        sem_w.at[NCH - 1])
    for c in range(NCH):
        wb[c].wait()


def kernel(X, pref_a, pref_b):
    mesh = plsc.VectorSubcoreMesh(core_axis_name="c", subcore_axis_name="s")
    k = pl.kernel(
        _sc_body,
        out_type=jax.ShapeDtypeStruct((B, D), jnp.float32),
        mesh=mesh,
        scratch_types=[
            pltpu.VMEM((ROWS_PER_W,), jnp.int32),
            pltpu.VMEM((ROWS_PER_W,), jnp.int32),
            pltpu.VMEM((CHUNKS[0], D), jnp.float32),
            pltpu.VMEM((CHUNKS[1], D), jnp.float32),
            pltpu.SemaphoreType.DMA,
            pltpu.SemaphoreType.DMA((NCH,)),
            pltpu.SemaphoreType.DMA((NCH,)),
        ],
    )
    return k(X, pref_a.astype(jnp.int32), pref_b.astype(jnp.int32))


# speedup vs baseline: 1.0372x; 1.0179x over previous
import jax
import jax.numpy as jnp
from jax import lax
from jax.experimental import pallas as pl
from jax.experimental.pallas import tpu as pltpu
from jax.experimental.pallas import tpu_sc as plsc

B = 16384
D = 128
L = 16

_info = plsc.get_sparse_core_info()
_NC = _info.num_cores
_NW = _NC * _info.num_subcores
ROWS_PER_W = B // _NW
CHUNK = 256
NCH = ROWS_PER_W // CHUNK
RPI = 2


def _sc_body(x_hbm, a_hbm, b_hbm, out_hbm,
             idx_a, idx_b, buf, sem_i, sem_g, sem_w):
    wid = lax.axis_index("s") * _NC + lax.axis_index("c")
    base = wid * ROWS_PER_W

    ci_a = pltpu.async_copy(a_hbm.at[pl.ds(base, ROWS_PER_W)], idx_a, sem_i)
    ci_b = pltpu.async_copy(b_hbm.at[pl.ds(base, ROWS_PER_W)], idx_b, sem_i)
    ci_a.wait()
    ga = [pltpu.async_copy(x_hbm.at[idx_a.at[pl.ds(c * CHUNK, CHUNK)]],
                           buf.at[c], sem_g.at[c])
          for c in range(NCH)]
    ci_b.wait()

    gb = [None] * NCH
    wb = [None] * NCH
    for c in range(NCH):
        ga[c].wait()

        def rows(t, carry, c=c):
            for r in range(RPI):
                for j in range(D // L):
                    i = t * RPI + r
                    sl = pl.ds(j * L, L)
                    buf[c, i, sl] = -buf[c, i, sl]
            return carry

        lax.fori_loop(0, CHUNK // RPI, rows, 0)

        gb[c] = pltpu.async_copy(x_hbm.at[idx_b.at[pl.ds(c * CHUNK, CHUNK)]],
                                 buf.at[c], sem_g.at[c], add=True)
        if c > 0:
            gb[c - 1].wait()
            wb[c - 1] = pltpu.async_copy(
                buf.at[c - 1], out_hbm.at[pl.ds(base + (c - 1) * CHUNK, CHUNK)],
                sem_w.at[c - 1])

    gb[NCH - 1].wait()
    wb[NCH - 1] = pltpu.async_copy(
        buf.at[NCH - 1], out_hbm.at[pl.ds(base + (NCH - 1) * CHUNK, CHUNK)],
        sem_w.at[NCH - 1])
    for c in range(NCH):
        wb[c].wait()


def kernel(X, pref_a, pref_b):
    mesh = plsc.VectorSubcoreMesh(core_axis_name="c", subcore_axis_name="s")
    k = pl.kernel(
        _sc_body,
        out_type=jax.ShapeDtypeStruct((B, D), jnp.float32),
        mesh=mesh,
        scratch_types=[
            pltpu.VMEM((ROWS_PER_W,), jnp.int32),
            pltpu.VMEM((ROWS_PER_W,), jnp.int32),
            pltpu.VMEM((NCH, CHUNK, D), jnp.float32),
            pltpu.SemaphoreType.DMA,
            pltpu.SemaphoreType.DMA((NCH,)),
            pltpu.SemaphoreType.DMA((NCH,)),
        ],
    )
    return k(X, pref_a.astype(jnp.int32), pref_b.astype(jnp.int32))


# speedup vs baseline: 1.0502x; 1.0126x over previous
import jax
import jax.numpy as jnp
from jax import lax
from jax.experimental import pallas as pl
from jax.experimental.pallas import tpu as pltpu
from jax.experimental.pallas import tpu_sc as plsc

B = 16384
D = 128
L = 16

_info = plsc.get_sparse_core_info()
_NC = _info.num_cores
_NW = _NC * _info.num_subcores
ROWS_PER_W = B // _NW
CHUNKS = (192, 320)
OFFS = (0, 192)
NCH = len(CHUNKS)
RPI = 2


def _sc_body(x_hbm, a_hbm, b_hbm, out_hbm,
             idx_a, idx_b, buf0, buf1, sem_i, sem_g, sem_w):
    bufs = (buf0, buf1)
    wid = lax.axis_index("s") * _NC + lax.axis_index("c")
    base = wid * ROWS_PER_W

    ci_a = pltpu.async_copy(a_hbm.at[pl.ds(base, ROWS_PER_W)], idx_a, sem_i)
    ci_b = pltpu.async_copy(b_hbm.at[pl.ds(base, ROWS_PER_W)], idx_b, sem_i)
    ci_a.wait()
    ga = [pltpu.async_copy(x_hbm.at[idx_a.at[pl.ds(OFFS[c], CHUNKS[c])]],
                           bufs[c], sem_g.at[c])
          for c in range(NCH)]
    ci_b.wait()

    gb = [None] * NCH
    wb = [None] * NCH
    for c in range(NCH):
        ga[c].wait()

        def rows(t, carry, c=c):
            for r in range(RPI):
                for j in range(D // L):
                    i = t * RPI + r
                    sl = pl.ds(j * L, L)
                    bufs[c][i, sl] = -bufs[c][i, sl]
            return carry

        lax.fori_loop(0, CHUNKS[c] // RPI, rows, 0)

        gb[c] = pltpu.async_copy(x_hbm.at[idx_b.at[pl.ds(OFFS[c], CHUNKS[c])]],
                                 bufs[c], sem_g.at[c], add=True)
        if c > 0:
            gb[c - 1].wait()
            wb[c - 1] = pltpu.async_copy(
                bufs[c - 1], out_hbm.at[pl.ds(base + OFFS[c - 1], CHUNKS[c - 1])],
                sem_w.at[c - 1])

    gb[NCH - 1].wait()
    wb[NCH - 1] = pltpu.async_copy(
        bufs[NCH - 1], out_hbm.at[pl.ds(base + OFFS[NCH - 1], CHUNKS[NCH - 1])],
        sem_w.at[NCH - 1])
    for c in range(NCH):
        wb[c].wait()


def kernel(X, pref_a, pref_b):
    mesh = plsc.VectorSubcoreMesh(core_axis_name="c", subcore_axis_name="s")
    k = pl.kernel(
        _sc_body,
        out_type=jax.ShapeDtypeStruct((B, D), jnp.float32),
        mesh=mesh,
        scratch_types=[
            pltpu.VMEM((ROWS_PER_W,), jnp.int32),
            pltpu.VMEM((ROWS_PER_W,), jnp.int32),
            pltpu.VMEM((CHUNKS[0], D), jnp.float32),
            pltpu.VMEM((CHUNKS[1], D), jnp.float32),
            pltpu.SemaphoreType.DMA,
            pltpu.SemaphoreType.DMA((NCH,)),
            pltpu.SemaphoreType.DMA((NCH,)),
        ],
    )
    return k(X, pref_a.astype(jnp.int32), pref_b.astype(jnp.int32))


# speedup vs baseline: 1.0564x; 1.0058x over previous
import jax
import jax.numpy as jnp
from jax import lax
from jax.experimental import pallas as pl
from jax.experimental.pallas import tpu as pltpu
from jax.experimental.pallas import tpu_sc as plsc

B = 16384
D = 128
L = 16

_info = plsc.get_sparse_core_info()
_NC = _info.num_cores
_NW = _NC * _info.num_subcores
ROWS_PER_W = B // _NW
CHUNKS = (96, 160, 256)
NCH = len(CHUNKS)
OFFS = tuple(sum(CHUNKS[:c]) for c in range(NCH))
RPI = 2


def _sc_body(x_hbm, a_hbm, b_hbm, out_hbm, *refs):
    bufs = refs[2:2 + NCH]
    idx_a, idx_b = refs[0], refs[1]
    sem_i, sem_g, sem_w = refs[2 + NCH], refs[3 + NCH], refs[4 + NCH]
    wid = lax.axis_index("s") * _NC + lax.axis_index("c")
    base = wid * ROWS_PER_W

    ci_a = pltpu.async_copy(a_hbm.at[pl.ds(base, ROWS_PER_W)], idx_a, sem_i)
    ci_b = pltpu.async_copy(b_hbm.at[pl.ds(base, ROWS_PER_W)], idx_b, sem_i)
    ci_a.wait()
    ga = [pltpu.async_copy(x_hbm.at[idx_a.at[pl.ds(OFFS[c], CHUNKS[c])]],
                           bufs[c], sem_g.at[c])
          for c in range(NCH)]
    ci_b.wait()

    gb = [None] * NCH
    wb = [None] * NCH
    for c in range(NCH):
        ga[c].wait()

        def rows(t, carry, c=c):
            for r in range(RPI):
                for j in range(D // L):
                    i = t * RPI + r
                    sl = pl.ds(j * L, L)
                    bufs[c][i, sl] = -bufs[c][i, sl]
            return carry

        lax.fori_loop(0, CHUNKS[c] // RPI, rows, 0)

        gb[c] = pltpu.async_copy(x_hbm.at[idx_b.at[pl.ds(OFFS[c], CHUNKS[c])]],
                                 bufs[c], sem_g.at[c], add=True)
        if c > 0:
            gb[c - 1].wait()
            wb[c - 1] = pltpu.async_copy(
                bufs[c - 1], out_hbm.at[pl.ds(base + OFFS[c - 1], CHUNKS[c - 1])],
                sem_w.at[c - 1])

    gb[NCH - 1].wait()
    wb[NCH - 1] = pltpu.async_copy(
        bufs[NCH - 1], out_hbm.at[pl.ds(base + OFFS[NCH - 1], CHUNKS[NCH - 1])],
        sem_w.at[NCH - 1])
    for c in range(NCH):
        wb[c].wait()


def kernel(X, pref_a, pref_b):
    mesh = plsc.VectorSubcoreMesh(core_axis_name="c", subcore_axis_name="s")
    k = pl.kernel(
        _sc_body,
        out_type=jax.ShapeDtypeStruct((B, D), jnp.float32),
        mesh=mesh,
        scratch_types=[
            pltpu.VMEM((ROWS_PER_W,), jnp.int32),
            pltpu.VMEM((ROWS_PER_W,), jnp.int32),
            *[pltpu.VMEM((CHUNKS[c], D), jnp.float32) for c in range(NCH)],
            pltpu.SemaphoreType.DMA,
            pltpu.SemaphoreType.DMA((NCH,)),
            pltpu.SemaphoreType.DMA((NCH,)),
        ],
    )
    return k(X, pref_a.astype(jnp.int32), pref_b.astype(jnp.int32))
